# baseline (speedup 1.0000x reference)
import functools

import jax
import jax.numpy as jnp
import numpy as np
from jax import lax
from jax.experimental import pallas as pl
from jax.experimental.pallas import tpu as pltpu
from jax.experimental.pallas import tpu_sc as plsc

N = 10000
E = 320000
D = 128
ED = 16
NEUR = 64
C = 8
B = D // C

NC = 2
NS = 16
NW = NC * NS
EW = E // NW
K = 80
CH = EW // K
RPT = 624
TAIL = N - NS * RPT
ZR = 48
CP = 208
G = 5
CHG = CH // G

_T1 = 1000
_G1 = N // _T1
_T3 = 384
_G3 = (N + _T3 - 1) // _T3
_CPAD = _G3 * _T3

_l = np.arange(D)
_i = np.arange(B)
_m = np.arange(D)
_S_np = ((_i[None, :, None] == (_l % B)[:, None, None])
         & ((_m // B)[None, None, :] == (_l // B)[:, None, None]))
_S = _S_np.reshape(D, B * D).astype(np.float32)


def _permute_w2(W2, b2):
    W2p = W2.reshape(NEUR, C, B, B).transpose(0, 2, 1, 3).reshape(NEUR, B * D)
    b2p = b2.reshape(C, B, B).transpose(1, 0, 2).reshape(1, B * D)
    return W2p, b2p


def _block_mm(x_in, ea_ref, W1_ref, b1_ref, W2p_ref, b2p_ref, S_ref, rows):
    h = jnp.maximum(
        jnp.dot(ea_ref[...], W1_ref[...], preferred_element_type=jnp.float32)
        + b1_ref[...], 0.0)
    wfull = jnp.dot(h, W2p_ref[...], preferred_element_type=jnp.float32) + b2p_ref[...]
    xb = jnp.dot(x_in, S_ref[...], preferred_element_type=jnp.float32)
    p = xb * wfull
    acc = jnp.zeros((rows, D), jnp.float32)
    for i in range(B):
        acc = acc + p[:, i * D:(i + 1) * D]
    return acc


def _phase1_body(x_ref, ea_ref, W1_ref, b1_ref, W2p_ref, b2p_ref, S_ref, out_ref):
    out_ref[...] = _block_mm(x_ref[...], ea_ref, W1_ref, b1_ref, W2p_ref,
                             b2p_ref, S_ref, _T1).astype(jnp.bfloat16)


def _phase3_body(parts_ref, cnt_ref, ea_ref, W1_ref, b1_ref, W2p_ref, b2p_ref,
                 S_ref, mixWT_ref, mixb_ref, out_ref):
    tot = jnp.sum(cnt_ref[...], axis=(0, 1)).reshape(1, _T3)
    cnt = jnp.transpose(tot)
    v = (parts_ref[0].astype(jnp.float32) + parts_ref[1].astype(jnp.float32)
         ) / jnp.maximum(cnt, 1.0)
    acc = _block_mm(v.astype(jnp.bfloat16), ea_ref, W1_ref, b1_ref, W2p_ref,
                    b2p_ref, S_ref, _T3)
    out_ref[...] = jnp.dot(acc, mixWT_ref[...],
                           preferred_element_type=jnp.float32) + mixb_ref[...]


def _sc_edge_body(kyv_hbm, src_hbm, dst_hbm, outp_hbm, outc_hbm,
                  src_v0, dst_v0, src_v1, dst_v1, rows_a, rows_b, cnt_v,
                  acc_sh, sem_a, sem_b, sem_i0, sem_i1):
    cid = lax.axis_index("c")
    sid = lax.axis_index("s")
    wid = sid * NC + cid

    def _zc(r, _):
        cnt_v[pl.ds(r * 16, 16)] = jnp.zeros((16,), jnp.float32)
        return 0
    lax.fori_loop(0, N // 16, _zc, 0)

    def _zr(r, _):
        def _zl(c, _):
            rows_a[r, pl.ds(c * 32, 32)] = jnp.zeros((32,), jnp.bfloat16)
            return 0
        lax.fori_loop(0, D // 32, _zl, 0)
        return 0
    lax.fori_loop(0, ZR, _zr, 0)
    base = sid * RPT
    for q in range(RPT // ZR):
        pltpu.sync_copy(rows_a.at[pl.ds(0, ZR)], acc_sh.at[pl.ds(base + q * ZR, ZR)])
    @pl.when(sid == NS - 1)
    def _():
        pltpu.sync_copy(rows_a.at[pl.ds(0, TAIL)], acc_sh.at[pl.ds(NS * RPT, TAIL)])
    plsc.subcore_barrier()

    ibufs = [(src_v0, dst_v0, sem_i0), (src_v1, dst_v1, sem_i1)]
    pltpu.async_copy(src_hbm.at[wid, pl.ds(0, CHG)], src_v0, sem_i0)
    pltpu.async_copy(dst_hbm.at[wid, pl.ds(0, CHG)], dst_v0, sem_i0)
    for g in range(G):
        src_v, dst_v, sem_i = ibufs[g % 2]
        pltpu.make_async_copy(src_hbm.at[wid, pl.ds(g * CHG, CHG)], src_v, sem_i).wait()
        pltpu.make_async_copy(dst_hbm.at[wid, pl.ds(g * CHG, CHG)], dst_v, sem_i).wait()
        if g + 1 < G:
            nsrc, ndst, nsem = ibufs[(g + 1) % 2]
            pltpu.async_copy(src_hbm.at[wid, pl.ds((g + 1) * CHG, CHG)], nsrc, nsem)
            pltpu.async_copy(dst_hbm.at[wid, pl.ds((g + 1) * CHG, CHG)], ndst, nsem)
        pltpu.async_copy(kyv_hbm.at[src_v.at[0]], rows_a, sem_a)

        def _step(j, _):
            even = lax.rem(j, 2) == 0
            @pl.when(jnp.logical_and(j + 1 < CHG, even))
            def _():
                pltpu.async_copy(kyv_hbm.at[src_v.at[j + 1]], rows_b, sem_b)
            @pl.when(jnp.logical_and(j + 1 < CHG, jnp.logical_not(even)))
            def _():
                pltpu.async_copy(kyv_hbm.at[src_v.at[j + 1]], rows_a, sem_a)
            for v in range(K // 16):
                ii = dst_v[j, pl.ds(v * 16, 16)]
                plsc.addupdate_scatter(cnt_v, [ii], jnp.ones((16,), jnp.float32))
            @pl.when(even)
            def _():
                pltpu.make_async_copy(kyv_hbm.at[src_v.at[j]], rows_a, sem_a).wait()
                pltpu.sync_copy(rows_a, acc_sh.at[dst_v.at[j]], add=True)
            @pl.when(jnp.logical_not(even))
            def _():
                pltpu.make_async_copy(kyv_hbm.at[src_v.at[j]], rows_b, sem_b).wait()
                pltpu.sync_copy(rows_b, acc_sh.at[dst_v.at[j]], add=True)
            return 0

        lax.fori_loop(0, CHG, _step, 0, unroll=False)
    plsc.subcore_barrier()

    for q in range(RPT // CP):
        pltpu.sync_copy(acc_sh.at[pl.ds(base + q * CP, CP)],
                        outp_hbm.at[cid, pl.ds(base + q * CP, CP)])
    @pl.when(sid == NS - 1)
    def _():
        pltpu.sync_copy(acc_sh.at[pl.ds(NS * RPT, TAIL)],
                        outp_hbm.at[cid, pl.ds(NS * RPT, TAIL)])
    pltpu.sync_copy(cnt_v, outc_hbm.at[pl.ds(wid * N, N)])


def _sc_edge_pass(kyv, src, dst):
    mesh = plsc.VectorSubcoreMesh(core_axis_name="c", subcore_axis_name="s")
    fn = pl.kernel(
        _sc_edge_body,
        out_type=(jax.ShapeDtypeStruct((NC, N, D), jnp.bfloat16),
                  jax.ShapeDtypeStruct((NW * N,), jnp.float32)),
        mesh=mesh,
        scratch_types=[
            pltpu.VMEM((CHG, K), jnp.int32),
            pltpu.VMEM((CHG, K), jnp.int32),
            pltpu.VMEM((CHG, K), jnp.int32),
            pltpu.VMEM((CHG, K), jnp.int32),
            pltpu.VMEM((K, D), jnp.bfloat16),
            pltpu.VMEM((K, D), jnp.bfloat16),
            pltpu.VMEM((N,), jnp.float32),
            pltpu.VMEM_SHARED((N, D), jnp.bfloat16),
            pltpu.SemaphoreType.DMA,
            pltpu.SemaphoreType.DMA,
            pltpu.SemaphoreType.DMA,
            pltpu.SemaphoreType.DMA,
        ],
        compiler_params=pltpu.CompilerParams(needs_layout_passes=False,
                                             use_tc_tiling_on_sc=False),
    )
    return fn(kyv, src, dst)


@jax.jit
def kernel(x, edge_index, edge_attr,
           k1_W1, k1_b1, k1_W2, k1_b2,
           k2_W1, k2_b1, k2_W2, k2_b2,
           mix_W, mix_b):
    W2p1, b2p1 = _permute_w2(k1_W2, k1_b2)
    W2p2, b2p2 = _permute_w2(k2_W2, k2_b2)
    S_bf = jnp.asarray(_S, dtype=jnp.bfloat16)
    src = edge_index[1].reshape(NW, CH, K)
    dst = edge_index[0].reshape(NW, CH, K)

    full = lambda s: pl.BlockSpec(s, lambda i: (0,) * len(s))
    kyv = pl.pallas_call(
        _phase1_body,
        grid=(_G1,),
        in_specs=[
            pl.BlockSpec((_T1, D), lambda i: (i, 0)),
            pl.BlockSpec((_T1, ED), lambda i: (i, 0)),
            full((ED, NEUR)),
            full((1, NEUR)),
            full((NEUR, B * D)),
            full((1, B * D)),
            full((D, B * D)),
        ],
        out_specs=pl.BlockSpec((_T1, D), lambda i: (i, 0)),
        out_shape=jax.ShapeDtypeStruct((N, D), jnp.bfloat16),
    )(x.astype(jnp.bfloat16), edge_attr, k1_W1, k1_b1.reshape(1, NEUR),
      W2p1, b2p1, S_bf)

    parts, cnts = _sc_edge_pass(kyv, src, dst)
    cnts = jnp.pad(cnts.reshape(NW, N), ((0, 0), (0, _CPAD - N))
                   ).reshape(NW, _G3, _T3).transpose(1, 0, 2)

    out = pl.pallas_call(
        _phase3_body,
        grid=(_G3,),
        in_specs=[
            pl.BlockSpec((NC, _T3, D), lambda i: (0, i, 0)),
            pl.BlockSpec((1, NW, _T3), lambda i: (i, 0, 0)),
            pl.BlockSpec((_T3, ED), lambda i: (i, 0)),
            full((ED, NEUR)),
            full((1, NEUR)),
            full((NEUR, B * D)),
            full((1, B * D)),
            full((D, B * D)),
            full((D, D)),
            full((1, D)),
        ],
        out_specs=pl.BlockSpec((_T3, D), lambda i: (i, 0)),
        out_shape=jax.ShapeDtypeStruct((N, D), jnp.float32),
    )(parts, cnts, edge_attr, k2_W1, k2_b1.reshape(1, NEUR), W2p2, b2p2, S_bf,
      mix_W.T, mix_b.reshape(1, D))
    return out


# speedup vs baseline: 1.0304x; 1.0304x over previous
import functools

import jax
import jax.numpy as jnp
import numpy as np
from jax import lax
from jax.experimental import pallas as pl
from jax.experimental.pallas import tpu as pltpu
from jax.experimental.pallas import tpu_sc as plsc

N = 10000
E = 320000
D = 128
ED = 16
NEUR = 64
C = 8
B = D // C

NC = 2
NS = 16
NW = NC * NS
EW = E // NW
K = 80
CH = EW // K
RPT = 624
TAIL = N - NS * RPT
ZR = 48
CP = 208
G = 5
CHG = CH // G

_T1 = 1000
_G1 = N // _T1
_T3 = 384
_G3 = (N + _T3 - 1) // _T3
_CPAD = _G3 * _T3

_l = np.arange(D)
_i = np.arange(B)
_m = np.arange(D)
_S_np = ((_i[None, :, None] == (_l % B)[:, None, None])
         & ((_m // B)[None, None, :] == (_l // B)[:, None, None]))
_S = _S_np.reshape(D, B * D).astype(np.float32)


def _permute_w2(W2, b2):
    W2p = W2.reshape(NEUR, C, B, B).transpose(0, 2, 1, 3).reshape(NEUR, B * D)
    b2p = b2.reshape(C, B, B).transpose(1, 0, 2).reshape(1, B * D)
    return W2p, b2p


def _block_mm(x_in, ea_ref, W1_ref, b1_ref, W2p_ref, b2p_ref, S_ref, rows):
    h = jnp.maximum(
        jnp.dot(ea_ref[...], W1_ref[...], preferred_element_type=jnp.float32)
        + b1_ref[...], 0.0)
    wfull = jnp.dot(h, W2p_ref[...], preferred_element_type=jnp.float32) + b2p_ref[...]
    xb = jnp.dot(x_in.astype(jnp.bfloat16), S_ref[...],
                 preferred_element_type=jnp.float32)
    p = xb * wfull
    acc = jnp.zeros((rows, D), jnp.float32)
    for i in range(B):
        acc = acc + p[:, i * D:(i + 1) * D]
    return acc


def _phase1_body(x_ref, ea_ref, W1_ref, b1_ref, W2p_ref, b2p_ref, S_ref, out_ref):
    out_ref[...] = _block_mm(x_ref[...], ea_ref, W1_ref, b1_ref, W2p_ref,
                             b2p_ref, S_ref, _T1).astype(jnp.bfloat16)


def _phase3_body(parts_ref, cnt_ref, ea_ref, W1_ref, b1_ref, W2p_ref, b2p_ref,
                 S_ref, mixWT_ref, mixb_ref, out_ref):
    tot = jnp.sum(cnt_ref[...], axis=(0, 1)).reshape(1, _T3)
    cnt = jnp.transpose(tot)
    v = (parts_ref[0].astype(jnp.float32) + parts_ref[1].astype(jnp.float32)
         ) / jnp.maximum(cnt, 1.0)
    acc = _block_mm(v, ea_ref, W1_ref, b1_ref, W2p_ref, b2p_ref, S_ref, _T3)
    out_ref[...] = jnp.dot(acc, mixWT_ref[...],
                           preferred_element_type=jnp.float32) + mixb_ref[...]


def _sc_edge_body(kyv_hbm, src_hbm, dst_hbm, outp_hbm, outc_hbm,
                  src_v0, dst_v0, src_v1, dst_v1, rows_a, rows_b, cnt_v,
                  acc_sh, sem_a, sem_b, sem_i0, sem_i1):
    cid = lax.axis_index("c")
    sid = lax.axis_index("s")
    wid = sid * NC + cid

    def _zc(r, _):
        cnt_v[pl.ds(r * 16, 16)] = jnp.zeros((16,), jnp.float32)
        return 0
    lax.fori_loop(0, N // 16, _zc, 0)

    def _zr(r, _):
        def _zl(c, _):
            rows_a[r, pl.ds(c * 32, 32)] = jnp.zeros((32,), jnp.bfloat16)
            return 0
        lax.fori_loop(0, D // 32, _zl, 0)
        return 0
    lax.fori_loop(0, ZR, _zr, 0)
    base = sid * RPT
    for q in range(RPT // ZR):
        pltpu.sync_copy(rows_a.at[pl.ds(0, ZR)], acc_sh.at[pl.ds(base + q * ZR, ZR)])
    @pl.when(sid == NS - 1)
    def _():
        pltpu.sync_copy(rows_a.at[pl.ds(0, TAIL)], acc_sh.at[pl.ds(NS * RPT, TAIL)])
    plsc.subcore_barrier()

    ibufs = [(src_v0, dst_v0, sem_i0), (src_v1, dst_v1, sem_i1)]
    pltpu.async_copy(src_hbm.at[wid, pl.ds(0, CHG)], src_v0, sem_i0)
    pltpu.async_copy(dst_hbm.at[wid, pl.ds(0, CHG)], dst_v0, sem_i0)
    for g in range(G):
        src_v, dst_v, sem_i = ibufs[g % 2]
        pltpu.make_async_copy(src_hbm.at[wid, pl.ds(g * CHG, CHG)], src_v, sem_i).wait()
        pltpu.make_async_copy(dst_hbm.at[wid, pl.ds(g * CHG, CHG)], dst_v, sem_i).wait()
        if g + 1 < G:
            nsrc, ndst, nsem = ibufs[(g + 1) % 2]
            pltpu.async_copy(src_hbm.at[wid, pl.ds((g + 1) * CHG, CHG)], nsrc, nsem)
            pltpu.async_copy(dst_hbm.at[wid, pl.ds((g + 1) * CHG, CHG)], ndst, nsem)
        pltpu.async_copy(kyv_hbm.at[src_v.at[0]], rows_a, sem_a)

        def _step(j, _):
            even = lax.rem(j, 2) == 0
            @pl.when(jnp.logical_and(j + 1 < CHG, even))
            def _():
                pltpu.async_copy(kyv_hbm.at[src_v.at[j + 1]], rows_b, sem_b)
            @pl.when(jnp.logical_and(j + 1 < CHG, jnp.logical_not(even)))
            def _():
                pltpu.async_copy(kyv_hbm.at[src_v.at[j + 1]], rows_a, sem_a)
            for v in range(K // 16):
                ii = dst_v[j, pl.ds(v * 16, 16)]
                plsc.addupdate_scatter(cnt_v, [ii], jnp.ones((16,), jnp.float32))
            @pl.when(even)
            def _():
                pltpu.make_async_copy(kyv_hbm.at[src_v.at[j]], rows_a, sem_a).wait()
                pltpu.sync_copy(rows_a, acc_sh.at[dst_v.at[j]], add=True)
            @pl.when(jnp.logical_not(even))
            def _():
                pltpu.make_async_copy(kyv_hbm.at[src_v.at[j]], rows_b, sem_b).wait()
                pltpu.sync_copy(rows_b, acc_sh.at[dst_v.at[j]], add=True)
            return 0

        lax.fori_loop(0, CHG, _step, 0, unroll=False)
    plsc.subcore_barrier()

    for q in range(RPT // CP):
        pltpu.sync_copy(acc_sh.at[pl.ds(base + q * CP, CP)],
                        outp_hbm.at[cid, pl.ds(base + q * CP, CP)])
    @pl.when(sid == NS - 1)
    def _():
        pltpu.sync_copy(acc_sh.at[pl.ds(NS * RPT, TAIL)],
                        outp_hbm.at[cid, pl.ds(NS * RPT, TAIL)])
    pltpu.sync_copy(cnt_v, outc_hbm.at[pl.ds(wid * N, N)])


def _sc_edge_pass(kyv, src, dst):
    mesh = plsc.VectorSubcoreMesh(core_axis_name="c", subcore_axis_name="s")
    fn = pl.kernel(
        _sc_edge_body,
        out_type=(jax.ShapeDtypeStruct((NC, N, D), jnp.bfloat16),
                  jax.ShapeDtypeStruct((NW * N,), jnp.float32)),
        mesh=mesh,
        scratch_types=[
            pltpu.VMEM((CHG, K), jnp.int32),
            pltpu.VMEM((CHG, K), jnp.int32),
            pltpu.VMEM((CHG, K), jnp.int32),
            pltpu.VMEM((CHG, K), jnp.int32),
            pltpu.VMEM((K, D), jnp.bfloat16),
            pltpu.VMEM((K, D), jnp.bfloat16),
            pltpu.VMEM((N,), jnp.float32),
            pltpu.VMEM_SHARED((N, D), jnp.bfloat16),
            pltpu.SemaphoreType.DMA,
            pltpu.SemaphoreType.DMA,
            pltpu.SemaphoreType.DMA,
            pltpu.SemaphoreType.DMA,
        ],
        compiler_params=pltpu.CompilerParams(needs_layout_passes=False,
                                             use_tc_tiling_on_sc=False),
    )
    return fn(kyv, src, dst)


@jax.jit
def kernel(x, edge_index, edge_attr,
           k1_W1, k1_b1, k1_W2, k1_b2,
           k2_W1, k2_b1, k2_W2, k2_b2,
           mix_W, mix_b):
    W2p1, b2p1 = _permute_w2(k1_W2, k1_b2)
    W2p2, b2p2 = _permute_w2(k2_W2, k2_b2)
    S_bf = jnp.asarray(_S, dtype=jnp.bfloat16)
    src = edge_index[1].reshape(NW, CH, K)
    dst = edge_index[0].reshape(NW, CH, K)

    full = lambda s: pl.BlockSpec(s, lambda i: (0,) * len(s))
    kyv = pl.pallas_call(
        _phase1_body,
        grid=(_G1,),
        in_specs=[
            pl.BlockSpec((_T1, D), lambda i: (i, 0)),
            pl.BlockSpec((_T1, ED), lambda i: (i, 0)),
            full((ED, NEUR)),
            full((1, NEUR)),
            full((NEUR, B * D)),
            full((1, B * D)),
            full((D, B * D)),
        ],
        out_specs=pl.BlockSpec((_T1, D), lambda i: (i, 0)),
        out_shape=jax.ShapeDtypeStruct((N, D), jnp.bfloat16),
    )(x, edge_attr, k1_W1, k1_b1.reshape(1, NEUR), W2p1, b2p1, S_bf)

    parts, cnts = _sc_edge_pass(kyv, src, dst)
    cnts = jnp.pad(cnts.reshape(NW, N), ((0, 0), (0, _CPAD - N))
                   ).reshape(NW, _G3, _T3).transpose(1, 0, 2)

    out = pl.pallas_call(
        _phase3_body,
        grid=(_G3,),
        in_specs=[
            pl.BlockSpec((NC, _T3, D), lambda i: (0, i, 0)),
            pl.BlockSpec((1, NW, _T3), lambda i: (i, 0, 0)),
            pl.BlockSpec((_T3, ED), lambda i: (i, 0)),
            full((ED, NEUR)),
            full((1, NEUR)),
            full((NEUR, B * D)),
            full((1, B * D)),
            full((D, B * D)),
            full((D, D)),
            full((1, D)),
        ],
        out_specs=pl.BlockSpec((_T3, D), lambda i: (i, 0)),
        out_shape=jax.ShapeDtypeStruct((N, D), jnp.float32),
    )(parts, cnts, edge_attr, k2_W1, k2_b1.reshape(1, NEUR), W2p2, b2p2, S_bf,
      mix_W.T, mix_b.reshape(1, D))
    return out
